# baseline (speedup 1.0000x reference)
import jax
import jax.numpy as jnp
from jax import lax
from jax.experimental import pallas as pl
from jax.experimental.pallas import tpu as pltpu
from jax.experimental.pallas import tpu_sc as plsc

N = 4096
D = 1024
H = 512
OUT = 1024
E = 8
TOPK = 2
LB_ALPHA = 0.01

A = 2 * N
B = 256
NB = A // B + E
P = NB * B
NPAD = P - A

NTILES = 32
RPT = P // NTILES
GCH = 32
NCH = RPT // GCH
TPT = N // NTILES
CCH = 16


def _gating_body(x_ref, wg_ref, bg_ref, p_ref, ga_ref, be_ref, loss_ref,
                 x16_ref):
    xb = x_ref[...]
    x16_ref[...] = xb.astype(jnp.bfloat16)
    logits = lax.dot_general(xb, wg_ref[...], (((1,), (1,)), ((), ())),
                             preferred_element_type=jnp.float32) + bg_ref[...]
    eidx = lax.broadcasted_iota(jnp.int32, (N, E), 1)
    m1 = jnp.max(logits, axis=1, keepdims=True)
    i1 = jnp.min(jnp.where(logits == m1, eidx, E), axis=1, keepdims=True)
    l2m = jnp.where(eidx == i1, -jnp.inf, logits)
    m2 = jnp.max(l2m, axis=1, keepdims=True)
    i2 = jnp.min(jnp.where(l2m == m2, eidx, E), axis=1, keepdims=True)

    r = jnp.exp(m2 - m1)
    den = 1.0 + r
    g1 = 1.0 / den
    g2 = r / den

    ex = jnp.exp(logits - m1)
    sm = ex / jnp.sum(ex, axis=1, keepdims=True)
    imp = jnp.sum(sm, axis=0, keepdims=True) * (1.0 / N)
    loss_ref[...] = jnp.sum(imp * imp, axis=1, keepdims=True) * (E * LB_ALPHA)

    e_a = jnp.concatenate([i1, i2], axis=0)
    oh = (e_a == lax.broadcasted_iota(jnp.int32, (A, E), 1)).astype(jnp.float32)
    cs = oh
    sh = 1
    while sh < A:
        cs = cs + jnp.concatenate(
            [jnp.zeros((sh, E), jnp.float32), cs[:A - sh]], axis=0)
        sh *= 2
    rank = cs - oh
    counts = cs[A - 1:A, :]
    blk = jnp.floor((counts + (B - 1)) * (1.0 / B))
    tri = (lax.broadcasted_iota(jnp.int32, (E, E), 0)
           < lax.broadcasted_iota(jnp.int32, (E, E), 1)).astype(jnp.float32)
    blkoff = lax.dot_general(blk, tri, (((1,), (0,)), ((), ())),
                             preferred_element_type=jnp.float32)
    rowoff = blkoff * B
    p_real = jnp.sum(oh * (rank + rowoff), axis=1, keepdims=True)

    padcnt = blk * B - counts
    padoff = lax.dot_general(padcnt, tri, (((1,), (0,)), ((), ())),
                             preferred_element_type=jnp.float32)
    padinc = padoff + padcnt
    jp = lax.broadcasted_iota(jnp.int32, (NPAD, E), 0)
    ejcnt = jnp.sum((padinc.astype(jnp.int32) <= jp).astype(jnp.int32),
                    axis=1, keepdims=True)
    ej = jnp.minimum(ejcnt, E - 1)
    ohp = (ej == lax.broadcasted_iota(jnp.int32, (NPAD, E), 1)).astype(
        jnp.float32)
    base = rowoff + counts - padoff
    p_pad = (jnp.sum(ohp * base, axis=1, keepdims=True)
             + lax.broadcasted_iota(jnp.int32, (NPAD, 1), 0).astype(
                 jnp.float32))
    p_ref[...] = jnp.concatenate([p_real, p_pad], axis=0).astype(jnp.int32)
    ga_ref[...] = jnp.concatenate(
        [g1, g2, jnp.zeros((NPAD, 1), jnp.float32)], axis=0)

    cumblk = (blkoff + blk).astype(jnp.int32)
    ii = lax.broadcasted_iota(jnp.int32, (NB, E), 0)
    becnt = jnp.sum((cumblk <= ii).astype(jnp.int32), axis=1, keepdims=True)
    be_ref[...] = jnp.minimum(becnt, E - 1)


def _gating_call(x, Wg, bg):
    return pl.pallas_call(
        _gating_body,
        out_shape=[
            jax.ShapeDtypeStruct((P, 1), jnp.int32),
            jax.ShapeDtypeStruct((P, 1), jnp.float32),
            jax.ShapeDtypeStruct((NB, 1), jnp.int32),
            jax.ShapeDtypeStruct((1, 1), jnp.float32),
            jax.ShapeDtypeStruct((N, D), jnp.bfloat16),
        ],
    )(x, Wg, bg.reshape(1, E))


def _dispatch_body(x_hbm, p3_hbm, xg_hbm, idx_v, rows0, rows1, sem):
    wid = lax.axis_index("s") * 2 + lax.axis_index("c")
    pltpu.sync_copy(p3_hbm.at[wid], idx_v)
    bufs = (rows0, rows1)
    prev = None
    for j in range(NCH):
        s = wid * RPT + j * GCH
        xs = pl.multiple_of(jnp.bitwise_and(s, N - 1), GCH)
        pltpu.sync_copy(x_hbm.at[pl.ds(xs, GCH)], bufs[j % 2])
        if prev is not None:
            prev.wait()
        prev = pltpu.async_copy(bufs[j % 2], xg_hbm.at[idx_v.at[j]], sem)
    prev.wait()


def _dispatch_call(x16w, p3):
    mesh = plsc.VectorSubcoreMesh(core_axis_name="c", subcore_axis_name="s")
    return pl.kernel(
        _dispatch_body,
        out_type=jax.ShapeDtypeStruct((P, D // 2), jnp.float32),
        mesh=mesh,
        scratch_types=[
            pltpu.VMEM((NCH, GCH), jnp.int32),
            pltpu.VMEM((GCH, D // 2), jnp.float32),
            pltpu.VMEM((GCH, D // 2), jnp.float32),
            pltpu.SemaphoreType.DMA,
        ],
        compiler_params=pltpu.CompilerParams(needs_layout_passes=False),
    )(x16w, p3)


def _ffn_body(be_ref, xg_ref, w1_ref, b1_ref, w2_ref, b2_ref, ys_ref):
    h = lax.dot_general(xg_ref[...], w1_ref[0], (((1,), (1,)), ((), ())),
                        preferred_element_type=jnp.float32,
                        precision=lax.Precision.DEFAULT)
    h = jnp.maximum(h + b1_ref[0], 0.0)
    y = lax.dot_general(h, w2_ref[0], (((1,), (1,)), ((), ())),
                        preferred_element_type=jnp.float32,
                        precision=lax.Precision.DEFAULT)
    ys_ref[...] = y + b2_ref[0]


def _ffn_call(be, xg, W1, b1, W2, b2):
    grid_spec = pltpu.PrefetchScalarGridSpec(
        num_scalar_prefetch=1,
        grid=(NB,),
        in_specs=[
            pl.BlockSpec((B, D), lambda i, be: (i, 0)),
            pl.BlockSpec((1, H, D), lambda i, be: (be[i], 0, 0)),
            pl.BlockSpec((1, 1, H), lambda i, be: (be[i], 0, 0)),
            pl.BlockSpec((1, OUT, H), lambda i, be: (be[i], 0, 0)),
            pl.BlockSpec((1, 1, OUT), lambda i, be: (be[i], 0, 0)),
        ],
        out_specs=pl.BlockSpec((B, OUT), lambda i, be: (i, 0)),
    )
    return pl.pallas_call(
        _ffn_body,
        grid_spec=grid_spec,
        out_shape=jax.ShapeDtypeStruct((P, OUT), jnp.float32),
        compiler_params=pltpu.CompilerParams(
            dimension_semantics=("arbitrary",)),
    )(be, xg, W1, b1.reshape(E, 1, H), W2, b2.reshape(E, 1, OUT))


def _combine_body(ys_hbm, p_hbm, ga_hbm, out_hbm,
                  idx1_v, idx2_v, g1_v, g2_v,
                  b1a, b2a, b1b, b2b, s1a, s2a, s1b, s2b, so_a, so_b):
    wid = lax.axis_index("s") * 2 + lax.axis_index("c")
    tbase = wid * TPT
    nchk = TPT // CCH
    pltpu.sync_copy(p_hbm.at[pl.ds(tbase, TPT)], idx1_v)
    pltpu.sync_copy(p_hbm.at[pl.ds(N + tbase, TPT)], idx2_v)
    pltpu.sync_copy(ga_hbm.at[pl.ds(tbase, TPT)], g1_v)
    pltpu.sync_copy(ga_hbm.at[pl.ds(N + tbase, TPT)], g2_v)
    b1s = (b1a, b1b)
    b2s = (b2a, b2b)
    s1s = (s1a, s1b)
    s2s = (s2a, s2b)
    sos = (so_a, so_b)

    def fire(j, pr):
        c1 = pltpu.async_copy(
            ys_hbm.at[idx1_v.at[pl.ds(j * CCH, CCH)]], b1s[pr], s1s[pr])
        c2 = pltpu.async_copy(
            ys_hbm.at[idx2_v.at[pl.ds(j * CCH, CCH)]], b2s[pr], s2s[pr])
        return c1, c2

    pend = {0: fire(0, 0)}
    outw = {}
    for j in range(nchk):
        pr = j & 1
        if j + 1 < nchk:
            if j - 1 in outw:
                outw.pop(j - 1).wait()
            pend[j + 1] = fire(j + 1, 1 - pr)
        c1, c2 = pend.pop(j)
        c1.wait()
        c2.wait()
        buf1 = b1s[pr]
        buf2 = b2s[pr]

        def rbody2(rr, c):
            gidx = lax.broadcast(j * CCH, (16,)) + lax.broadcast(rr, (16,))
            g1s = plsc.load_gather(g1_v, [gidx])
            g2s = plsc.load_gather(g2_v, [gidx])
            for k in range(OUT // 16):
                col = k * 16
                buf1[rr, pl.ds(col, 16)] = (
                    g1s * buf1[rr, pl.ds(col, 16)]
                    + g2s * buf2[rr, pl.ds(col, 16)])
            return c
        lax.fori_loop(0, CCH, rbody2, 0)
        outw[j] = pltpu.async_copy(
            buf1, out_hbm.at[pl.ds(tbase + j * CCH, CCH)], sos[pr])
    for j in list(outw):
        outw.pop(j).wait()


def _combine_call(ys, p, ga):
    mesh = plsc.VectorSubcoreMesh(core_axis_name="c", subcore_axis_name="s")
    return pl.kernel(
        _combine_body,
        out_type=jax.ShapeDtypeStruct((N, OUT), jnp.float32),
        mesh=mesh,
        scratch_types=[
            pltpu.VMEM((TPT,), jnp.int32),
            pltpu.VMEM((TPT,), jnp.int32),
            pltpu.VMEM((TPT,), jnp.float32),
            pltpu.VMEM((TPT,), jnp.float32),
            pltpu.VMEM((CCH, OUT), jnp.float32),
            pltpu.VMEM((CCH, OUT), jnp.float32),
            pltpu.VMEM((CCH, OUT), jnp.float32),
            pltpu.VMEM((CCH, OUT), jnp.float32),
            pltpu.SemaphoreType.DMA,
            pltpu.SemaphoreType.DMA,
            pltpu.SemaphoreType.DMA,
            pltpu.SemaphoreType.DMA,
            pltpu.SemaphoreType.DMA,
            pltpu.SemaphoreType.DMA,
        ],
        compiler_params=pltpu.CompilerParams(needs_layout_passes=False),
    )(ys, p, ga)


def kernel(x, Wg, bg, W1, b1, W2, b2):
    p2d, ga2d, be2d, loss11, x16 = _gating_call(x, Wg, bg)
    p = p2d.reshape(P)
    ga = ga2d.reshape(P)
    be = be2d.reshape(NB)
    x16w = lax.bitcast_convert_type(
        x16.reshape(N, D // 2, 2), jnp.float32)
    xgw = _dispatch_call(x16w, p.reshape(NTILES, NCH, GCH))
    xg = lax.bitcast_convert_type(xgw, jnp.bfloat16).reshape(P, D)
    ys = _ffn_call(be, xg, W1, b1, W2, b2)
    out = _combine_call(ys, p[:A], ga[:A])
    return (out, loss11.reshape(()))


# speedup vs baseline: 2.6998x; 2.6998x over previous
import jax
import jax.numpy as jnp
from jax import lax
from jax.experimental import pallas as pl
from jax.experimental.pallas import tpu as pltpu
from jax.experimental.pallas import tpu_sc as plsc

N = 4096
D = 1024
H = 512
OUT = 1024
E = 8
TOPK = 2
LB_ALPHA = 0.01

A = 2 * N
B = 256
NB = A // B + E
P = NB * B
NPAD = P - A

NTILES = 32
RPT = P // NTILES
GCH = 32
NCH = RPT // GCH
TPT = N // NTILES
CCH = 16


def _gating_body(x_ref, wg_ref, bg_ref, p_ref, ga_ref, be_ref, loss_ref):
    xb = x_ref[...]
    logits = lax.dot_general(xb, wg_ref[...], (((1,), (1,)), ((), ())),
                             preferred_element_type=jnp.float32) + bg_ref[...]
    eidx = lax.broadcasted_iota(jnp.int32, (N, E), 1)
    m1 = jnp.max(logits, axis=1, keepdims=True)
    i1 = jnp.min(jnp.where(logits == m1, eidx, E), axis=1, keepdims=True)
    l2m = jnp.where(eidx == i1, -jnp.inf, logits)
    m2 = jnp.max(l2m, axis=1, keepdims=True)
    i2 = jnp.min(jnp.where(l2m == m2, eidx, E), axis=1, keepdims=True)

    r = jnp.exp(m2 - m1)
    den = 1.0 + r
    g1 = 1.0 / den
    g2 = r / den

    ex = jnp.exp(logits - m1)
    sm = ex / jnp.sum(ex, axis=1, keepdims=True)
    imp = jnp.sum(sm, axis=0, keepdims=True) * (1.0 / N)
    loss_ref[...] = jnp.sum(imp * imp, axis=1, keepdims=True) * (E * LB_ALPHA)

    e_a = jnp.concatenate([i1, i2], axis=0)
    oh = (e_a == lax.broadcasted_iota(jnp.int32, (A, E), 1)).astype(jnp.float32)
    cs = oh
    sh = 1
    while sh < A:
        cs = cs + jnp.concatenate(
            [jnp.zeros((sh, E), jnp.float32), cs[:A - sh]], axis=0)
        sh *= 2
    rank = cs - oh
    counts = cs[A - 1:A, :]
    blk = jnp.floor((counts + (B - 1)) * (1.0 / B))
    tri = (lax.broadcasted_iota(jnp.int32, (E, E), 0)
           < lax.broadcasted_iota(jnp.int32, (E, E), 1)).astype(jnp.float32)
    blkoff = lax.dot_general(blk, tri, (((1,), (0,)), ((), ())),
                             preferred_element_type=jnp.float32)
    rowoff = blkoff * B
    p_real = jnp.sum(oh * (rank + rowoff), axis=1, keepdims=True)

    padcnt = blk * B - counts
    padoff = lax.dot_general(padcnt, tri, (((1,), (0,)), ((), ())),
                             preferred_element_type=jnp.float32)
    padinc = padoff + padcnt
    jp = lax.broadcasted_iota(jnp.int32, (NPAD, E), 0)
    ejcnt = jnp.sum((padinc.astype(jnp.int32) <= jp).astype(jnp.int32),
                    axis=1, keepdims=True)
    ej = jnp.minimum(ejcnt, E - 1)
    ohp = (ej == lax.broadcasted_iota(jnp.int32, (NPAD, E), 1)).astype(
        jnp.float32)
    base = rowoff + counts - padoff
    p_pad = (jnp.sum(ohp * base, axis=1, keepdims=True)
             + lax.broadcasted_iota(jnp.int32, (NPAD, 1), 0).astype(
                 jnp.float32))
    p_ref[...] = jnp.concatenate([p_real, p_pad], axis=0).astype(jnp.int32)
    ga_ref[...] = jnp.concatenate(
        [g1, g2, jnp.zeros((NPAD, 1), jnp.float32)], axis=0)

    cumblk = (blkoff + blk).astype(jnp.int32)
    ii = lax.broadcasted_iota(jnp.int32, (NB, E), 0)
    becnt = jnp.sum((cumblk <= ii).astype(jnp.int32), axis=1, keepdims=True)
    be_ref[...] = jnp.minimum(becnt, E - 1)


def _gating_call(x, Wg, bg):
    return pl.pallas_call(
        _gating_body,
        out_shape=[
            jax.ShapeDtypeStruct((P, 1), jnp.int32),
            jax.ShapeDtypeStruct((P, 1), jnp.float32),
            jax.ShapeDtypeStruct((NB, 1), jnp.int32),
            jax.ShapeDtypeStruct((1, 1), jnp.float32),
        ],
    )(x, Wg, bg.reshape(1, E))


def _dispatch_body(x_hbm, p3_hbm, xg_hbm, idx_v, rows0, rows1, sem):
    wid = lax.axis_index("s") * 2 + lax.axis_index("c")
    pltpu.sync_copy(p3_hbm.at[wid], idx_v)
    bufs = (rows0, rows1)
    prev = None
    for j in range(NCH):
        s = wid * RPT + j * GCH
        xs = pl.multiple_of(jnp.bitwise_and(s, N - 1), GCH)
        pltpu.sync_copy(x_hbm.at[pl.ds(xs, GCH)], bufs[j % 2])
        if prev is not None:
            prev.wait()
        prev = pltpu.async_copy(bufs[j % 2], xg_hbm.at[idx_v.at[j]], sem)
    prev.wait()


def _dispatch_call(x, p3):
    mesh = plsc.VectorSubcoreMesh(core_axis_name="c", subcore_axis_name="s")
    return pl.kernel(
        _dispatch_body,
        out_type=jax.ShapeDtypeStruct((P, D), jnp.float32),
        mesh=mesh,
        scratch_types=[
            pltpu.VMEM((NCH, GCH), jnp.int32),
            pltpu.VMEM((GCH, D), jnp.float32),
            pltpu.VMEM((GCH, D), jnp.float32),
            pltpu.SemaphoreType.DMA,
        ],
        compiler_params=pltpu.CompilerParams(needs_layout_passes=False),
    )(x, p3)


def _ffn_body(be_ref, xg_ref, w1_ref, b1_ref, w2_ref, b2_ref, ys_ref):
    h = lax.dot_general(xg_ref[...], w1_ref[0], (((1,), (1,)), ((), ())),
                        preferred_element_type=jnp.float32,
                        precision=lax.Precision.DEFAULT)
    h = jnp.maximum(h + b1_ref[0], 0.0)
    y = lax.dot_general(h, w2_ref[0], (((1,), (1,)), ((), ())),
                        preferred_element_type=jnp.float32,
                        precision=lax.Precision.DEFAULT)
    ys_ref[...] = y + b2_ref[0]


def _ffn_call(be, xg, W1, b1, W2, b2):
    grid_spec = pltpu.PrefetchScalarGridSpec(
        num_scalar_prefetch=1,
        grid=(NB,),
        in_specs=[
            pl.BlockSpec((B, D), lambda i, be: (i, 0)),
            pl.BlockSpec((1, H, D), lambda i, be: (be[i], 0, 0)),
            pl.BlockSpec((1, 1, H), lambda i, be: (be[i], 0, 0)),
            pl.BlockSpec((1, OUT, H), lambda i, be: (be[i], 0, 0)),
            pl.BlockSpec((1, 1, OUT), lambda i, be: (be[i], 0, 0)),
        ],
        out_specs=pl.BlockSpec((B, OUT), lambda i, be: (i, 0)),
    )
    return pl.pallas_call(
        _ffn_body,
        grid_spec=grid_spec,
        out_shape=jax.ShapeDtypeStruct((P, OUT), jnp.float32),
        compiler_params=pltpu.CompilerParams(
            dimension_semantics=("arbitrary",)),
    )(be, xg, W1, b1.reshape(E, 1, H), W2, b2.reshape(E, 1, OUT))


def _combine_body(ys_hbm, p_hbm, ga_hbm, out_hbm,
                  idx1_v, idx2_v, g1_v, g2_v,
                  b1a, b2a, b1b, b2b, s1a, s2a, s1b, s2b, so_a, so_b):
    wid = lax.axis_index("s") * 2 + lax.axis_index("c")
    tbase = wid * TPT
    nchk = TPT // CCH
    pltpu.sync_copy(p_hbm.at[pl.ds(tbase, TPT)], idx1_v)
    pltpu.sync_copy(p_hbm.at[pl.ds(N + tbase, TPT)], idx2_v)
    pltpu.sync_copy(ga_hbm.at[pl.ds(tbase, TPT)], g1_v)
    pltpu.sync_copy(ga_hbm.at[pl.ds(N + tbase, TPT)], g2_v)
    b1s = (b1a, b1b)
    b2s = (b2a, b2b)
    s1s = (s1a, s1b)
    s2s = (s2a, s2b)
    sos = (so_a, so_b)

    def fire(j, pr):
        c1 = pltpu.async_copy(
            ys_hbm.at[idx1_v.at[pl.ds(j * CCH, CCH)]], b1s[pr], s1s[pr])
        c2 = pltpu.async_copy(
            ys_hbm.at[idx2_v.at[pl.ds(j * CCH, CCH)]], b2s[pr], s2s[pr])
        return c1, c2

    pend = {0: fire(0, 0)}
    outw = {}
    for j in range(nchk):
        pr = j & 1
        if j + 1 < nchk:
            if j - 1 in outw:
                outw.pop(j - 1).wait()
            pend[j + 1] = fire(j + 1, 1 - pr)
        c1, c2 = pend.pop(j)
        c1.wait()
        c2.wait()
        buf1 = b1s[pr]
        buf2 = b2s[pr]

        def rbody2(rr, c):
            gidx = lax.broadcast(j * CCH, (16,)) + lax.broadcast(rr, (16,))
            g1s = plsc.load_gather(g1_v, [gidx])
            g2s = plsc.load_gather(g2_v, [gidx])
            for k in range(OUT // 16):
                col = k * 16
                buf1[rr, pl.ds(col, 16)] = (
                    g1s * buf1[rr, pl.ds(col, 16)]
                    + g2s * buf2[rr, pl.ds(col, 16)])
            return c
        lax.fori_loop(0, CCH, rbody2, 0)
        outw[j] = pltpu.async_copy(
            buf1, out_hbm.at[pl.ds(tbase + j * CCH, CCH)], sos[pr])
    for j in list(outw):
        outw.pop(j).wait()


def _combine_call(ys, p, ga):
    mesh = plsc.VectorSubcoreMesh(core_axis_name="c", subcore_axis_name="s")
    return pl.kernel(
        _combine_body,
        out_type=jax.ShapeDtypeStruct((N, OUT), jnp.float32),
        mesh=mesh,
        scratch_types=[
            pltpu.VMEM((TPT,), jnp.int32),
            pltpu.VMEM((TPT,), jnp.int32),
            pltpu.VMEM((TPT,), jnp.float32),
            pltpu.VMEM((TPT,), jnp.float32),
            pltpu.VMEM((CCH, OUT), jnp.float32),
            pltpu.VMEM((CCH, OUT), jnp.float32),
            pltpu.VMEM((CCH, OUT), jnp.float32),
            pltpu.VMEM((CCH, OUT), jnp.float32),
            pltpu.SemaphoreType.DMA,
            pltpu.SemaphoreType.DMA,
            pltpu.SemaphoreType.DMA,
            pltpu.SemaphoreType.DMA,
            pltpu.SemaphoreType.DMA,
            pltpu.SemaphoreType.DMA,
        ],
        compiler_params=pltpu.CompilerParams(needs_layout_passes=False),
    )(ys, p, ga)


def kernel(x, Wg, bg, W1, b1, W2, b2):
    p2d, ga2d, be2d, loss11 = _gating_call(x, Wg, bg)
    p = p2d.reshape(P)
    ga = ga2d.reshape(P)
    be = be2d.reshape(NB)
    xg = _dispatch_call(x, p.reshape(NTILES, NCH, GCH))
    ys = _ffn_call(be, xg, W1, b1, W2, b2)
    out = _combine_call(ys, p[:A], ga[:A])
    return (out, loss11.reshape(()))
